# baseline (speedup 1.0000x reference)
import functools

import jax
import jax.numpy as jnp
from jax import lax
from jax.experimental import pallas as pl
from jax.experimental.pallas import tpu as pltpu
from jax.experimental.pallas import tpu_sc as plsc

_N = 16384
_HALF = _N // 2
_BLK = 2048
_NW = 32
_CHUNK = _HALF // _NW


def _tc_body(a_ref, b_ref, w_ref, bias_ref, pa_ref, pb_ref):
    wt = w_ref[...]
    bias = bias_ref[...]
    laT = lax.dot_general(wt, a_ref[...], (((1,), (0,)), ((), ())),
                          preferred_element_type=jnp.float32) + bias
    lbT = lax.dot_general(wt, b_ref[...], (((1,), (0,)), ((), ())),
                          preferred_element_type=jnp.float32) + bias

    iota = lax.broadcasted_iota(jnp.int32, laT.shape, 0)
    pad = jnp.zeros((5, laT.shape[1]), jnp.float32)

    def softmax_argmax(logits):
        m = jnp.max(logits, axis=0, keepdims=True)
        e = jnp.exp(logits - m)
        p = e / jnp.sum(e, axis=0, keepdims=True)
        idx = jnp.min(jnp.where(logits == m, iota, 10), axis=0, keepdims=True)
        return jnp.concatenate([p, idx.astype(jnp.float32), pad], axis=0)

    pa_ref[...] = softmax_argmax(laT)
    pb_ref[...] = softmax_argmax(lbT)


def _tc_half(aT, bT, wT, bias_col, half):
    off = half * (_HALF // _BLK)
    return pl.pallas_call(
        _tc_body,
        grid=(_HALF // _BLK,),
        in_specs=[
            pl.BlockSpec((784, _BLK), lambda i: (0, i + off)),
            pl.BlockSpec((784, _BLK), lambda i: (0, i + off)),
            pl.BlockSpec((10, 784), lambda i: (0, 0)),
            pl.BlockSpec((10, 1), lambda i: (0, 0)),
        ],
        out_specs=[
            pl.BlockSpec((16, _BLK), lambda i: (0, i)),
            pl.BlockSpec((16, _BLK), lambda i: (0, i)),
        ],
        out_shape=[
            jax.ShapeDtypeStruct((16, _HALF), jnp.float32),
            jax.ShapeDtypeStruct((16, _HALF), jnp.float32),
        ],
        compiler_params=pltpu.CompilerParams(
            dimension_semantics=("parallel",),
        ),
    )(aT, bT, wT, bias_col)


@functools.partial(
    pl.kernel,
    out_type=jax.ShapeDtypeStruct((_HALF, 19), jnp.float32),
    mesh=plsc.VectorSubcoreMesh(core_axis_name="c", subcore_axis_name="s"),
    scratch_types=[
        pltpu.VMEM((16, _CHUNK), jnp.float32),
        pltpu.VMEM((16, _CHUNK), jnp.float32),
        pltpu.VMEM((_CHUNK, 19), jnp.float32),
        pltpu.SemaphoreType.DMA,
        pltpu.SemaphoreType.DMA,
    ],
    compiler_params=pltpu.CompilerParams(needs_layout_passes=False),
)
def _sc_join(a_hbm, b_hbm, out_hbm, a_v, b_v, s_v, sem_a, sem_b):
    wid = lax.axis_index("s") * 2 + lax.axis_index("c")
    base = wid * _CHUNK
    cp_a = pltpu.make_async_copy(a_hbm.at[:, pl.ds(base, _CHUNK)], a_v, sem_a)
    cp_b = pltpu.make_async_copy(b_hbm.at[:, pl.ds(base, _CHUNK)], b_v, sem_b)
    cp_a.start()
    cp_b.start()
    cp_a.wait()
    cp_b.wait()
    lane = lax.iota(jnp.int32, 16)

    def group(g, carry):
        col = g * 16
        row = col + lane
        a_cols = [a_v[i, pl.ds(col, 16)] for i in range(10)]
        b_cols = [b_v[j, pl.ds(col, 16)] for j in range(10)]
        bins = [None] * 19
        for i in range(10):
            for j in range(10):
                p = a_cols[i] * b_cols[j]
                k = i + j
                bins[k] = p if bins[k] is None else bins[k] + p
        for k in range(19):
            plsc.store_scatter(s_v, [row, jnp.full((16,), k, jnp.int32)],
                               bins[k])
        return carry

    lax.fori_loop(0, _CHUNK // 16, group, 0)
    pltpu.sync_copy(s_v, out_hbm.at[pl.ds(base, _CHUNK)])


@jax.jit
def _run(aT, bT, wT, bias_col):
    pa0, pb0 = _tc_half(aT, bT, wT, bias_col, 0)
    sp0 = _sc_join(pa0, pb0)
    pa1, pb1 = _tc_half(aT, bT, wT, bias_col, 1)
    sp1 = _sc_join(pa1, pb1)
    sp = jnp.concatenate([sp0, sp1], axis=0)
    ap = jnp.concatenate([pa0[10], pa1[10]]).astype(jnp.int32)
    bp = jnp.concatenate([pb0[10], pb1[10]]).astype(jnp.int32)
    return sp, ap, bp


def kernel(a_imgs, b_imgs, W, b):
    return _run(a_imgs.T, b_imgs.T, W.T, b.reshape(10, 1))


# speedup vs baseline: 1.2782x; 1.2782x over previous
import functools

import jax
import jax.numpy as jnp
from jax import lax
from jax.experimental import pallas as pl
from jax.experimental.pallas import tpu as pltpu
from jax.experimental.pallas import tpu_sc as plsc

_N = 16384
_BLK = 1024
_NW = 32
_CHUNK = _N // _NW


def _tc_body(a_ref, b_ref, w_ref, bias_ref, pa_ref, pb_ref):
    wt = w_ref[...]
    bias = bias_ref[...]
    laT = lax.dot_general(wt, a_ref[...], (((1,), (0,)), ((), ())),
                          preferred_element_type=jnp.float32) + bias
    lbT = lax.dot_general(wt, b_ref[...], (((1,), (0,)), ((), ())),
                          preferred_element_type=jnp.float32) + bias

    iota = lax.broadcasted_iota(jnp.int32, laT.shape, 0)
    pad = jnp.zeros((5, laT.shape[1]), jnp.float32)

    def softmax_argmax(logits):
        m = jnp.max(logits, axis=0, keepdims=True)
        e = jnp.exp(logits - m)
        p = e / jnp.sum(e, axis=0, keepdims=True)
        idx = jnp.min(jnp.where(logits == m, iota, 10), axis=0, keepdims=True)
        return jnp.concatenate([p, idx.astype(jnp.float32), pad], axis=0)

    pa_ref[...] = softmax_argmax(laT)
    pb_ref[...] = softmax_argmax(lbT)


@functools.partial(
    pl.kernel,
    out_type=jax.ShapeDtypeStruct((19, _N), jnp.float32),
    mesh=plsc.VectorSubcoreMesh(core_axis_name="c", subcore_axis_name="s"),
    scratch_types=[
        pltpu.VMEM((16, _CHUNK), jnp.float32),
        pltpu.VMEM((16, _CHUNK), jnp.float32),
        pltpu.VMEM((19, _CHUNK), jnp.float32),
        pltpu.SemaphoreType.DMA,
        pltpu.SemaphoreType.DMA,
    ],
    compiler_params=pltpu.CompilerParams(needs_layout_passes=False),
)
def _sc_join(a_hbm, b_hbm, out_hbm, a_v, b_v, s_v, sem_a, sem_b):
    wid = lax.axis_index("s") * 2 + lax.axis_index("c")
    base = wid * _CHUNK
    cp_a = pltpu.make_async_copy(a_hbm.at[:, pl.ds(base, _CHUNK)], a_v, sem_a)
    cp_b = pltpu.make_async_copy(b_hbm.at[:, pl.ds(base, _CHUNK)], b_v, sem_b)
    cp_a.start()
    cp_b.start()
    cp_a.wait()
    cp_b.wait()

    def group(g, carry):
        col = g * 16
        a_cols = [a_v[i, pl.ds(col, 16)] for i in range(10)]
        b_cols = [b_v[j, pl.ds(col, 16)] for j in range(10)]
        bins = [None] * 19
        for i in range(10):
            for j in range(10):
                p = a_cols[i] * b_cols[j]
                k = i + j
                bins[k] = p if bins[k] is None else bins[k] + p
        for k in range(19):
            s_v[k, pl.ds(col, 16)] = bins[k]
        return carry

    lax.fori_loop(0, _CHUNK // 16, group, 0)
    pltpu.sync_copy(s_v, out_hbm.at[:, pl.ds(base, _CHUNK)])


@jax.jit
def _run(aT, bT, wT, bias_col):
    grid = (_N // _BLK,)
    pa, pb = pl.pallas_call(
        _tc_body,
        grid=grid,
        in_specs=[
            pl.BlockSpec((784, _BLK), lambda i: (0, i)),
            pl.BlockSpec((784, _BLK), lambda i: (0, i)),
            pl.BlockSpec((10, 784), lambda i: (0, 0)),
            pl.BlockSpec((10, 1), lambda i: (0, 0)),
        ],
        out_specs=[
            pl.BlockSpec((16, _BLK), lambda i: (0, i)),
            pl.BlockSpec((16, _BLK), lambda i: (0, i)),
        ],
        out_shape=[
            jax.ShapeDtypeStruct((16, _N), jnp.float32),
            jax.ShapeDtypeStruct((16, _N), jnp.float32),
        ],
        compiler_params=pltpu.CompilerParams(
            dimension_semantics=("parallel",),
        ),
    )(aT, bT, wT, bias_col)
    spT = _sc_join(pa, pb)
    ap = pa[10].astype(jnp.int32)
    bp = pb[10].astype(jnp.int32)
    return spT.T, ap, bp


def kernel(a_imgs, b_imgs, W, b):
    return _run(a_imgs.T, b_imgs.T, W.T, b.reshape(10, 1))
